# baseline (speedup 1.0000x reference)
import functools

import jax
import jax.numpy as jnp
from jax import lax
from jax.experimental import pallas as pl
from jax.experimental.pallas import tpu as pltpu
from jax.experimental.pallas import tpu_sc as plsc

_L = 16


@functools.lru_cache(maxsize=None)
def _build_gather(V: int, D: int, B: int):
    info = plsc.get_sparse_core_info()
    NC, NS = info.num_cores, info.num_subcores
    NW = NC * NS
    assert B % NW == 0
    b_per_w = B // NW

    mesh = plsc.VectorSubcoreMesh(core_axis_name="c", subcore_axis_name="s")

    @functools.partial(
        pl.kernel,
        mesh=mesh,
        out_type=jax.ShapeDtypeStruct((B, D), jnp.float32),
        scratch_types=[
            pltpu.VMEM((b_per_w,), jnp.int32),
            pltpu.SemaphoreType.DMA,
        ],
        compiler_params=pltpu.CompilerParams(needs_layout_passes=False),
    )
    def gather(x_hbm, idx_hbm, out_hbm, idx_v, sem):
        wid = lax.axis_index("s") * NC + lax.axis_index("c")
        base = wid * b_per_w
        pltpu.sync_copy(idx_hbm.at[pl.ds(base, b_per_w)], idx_v)
        lanes = lax.iota(jnp.int32, _L)

        @pl.loop(0, b_per_w // _L)
        def _fire(g):
            vec = idx_v[pl.ds(g * _L, _L)]
            for l in range(_L):
                r = jnp.sum(jnp.where(lanes == l, vec, 0))
                pltpu.async_copy(
                    x_hbm.at[r], out_hbm.at[base + g * _L + l], sem)

        @pl.loop(0, b_per_w)
        def _drain(j):
            pltpu.make_async_copy(
                x_hbm.at[0], out_hbm.at[base], sem).wait()

    return gather


def kernel(x, selected_rows):
    V, D = x.shape
    B = selected_rows.shape[0]
    return _build_gather(V, D, B)(x, selected_rows.astype(jnp.int32))
